# baseline (speedup 1.0000x reference)
import functools

import jax
import jax.numpy as jnp
from jax import lax
from jax.experimental import pallas as pl
from jax.experimental.pallas import tpu as pltpu
from jax.experimental.pallas import tpu_sc as plsc

_B, _N, _M, _C, _O, _K = 8, 4096, 1024, 128, 128, 16
_RADIUS = 10.0
_PD = 16
_NC, _NS = 2, 16
_NW = _NC * _NS
_CH = 128


def _sc_mesh():
    return plsc.VectorSubcoreMesh(
        core_axis_name="c", subcore_axis_name="s",
        num_cores=_NC, num_subcores=_NS)


def _gather_rows_sc(table2d, idx_flat, width, rows_per_batch):
    rows = idx_flat.shape[0]
    rows_per_w = rows // _NW
    nch = rows_per_w // _CH

    @functools.partial(
        pl.kernel,
        out_type=jax.ShapeDtypeStruct((rows, width), jnp.float32),
        mesh=_sc_mesh(),
        scratch_types=[
            pltpu.VMEM((_CH,), jnp.int32),
            pltpu.VMEM((_CH, width), jnp.float32),
            pltpu.SemaphoreType.DMA,
        ],
    )
    def k(tab_hbm, idx_hbm, out_hbm, idx_v, rows_v, sem):
        wid = lax.axis_index("s") * _NC + lax.axis_index("c")
        boff = (wid * rows_per_w // rows_per_batch) * _N
        for t in range(nch):
            base = wid * rows_per_w + t * _CH
            pltpu.sync_copy(idx_hbm.at[pl.ds(base, _CH)], idx_v)
            for j in range(_CH // 16):
                sl = pl.ds(j * 16, 16)
                idx_v[sl] = idx_v[sl] + boff
            pltpu.async_copy(tab_hbm.at[idx_v], rows_v, sem).wait()
            pltpu.sync_copy(rows_v, out_hbm.at[pl.ds(base, _CH)])

    return k(table2d, idx_flat)


def _full_extract(vals, fiota):
    cols = []
    for _ in range(_K):
        m = jnp.min(vals, axis=1, keepdims=True)
        im = jnp.min(jnp.where(vals == m, fiota, jnp.float32(_N)),
                     axis=1, keepdims=True)
        cols.append(im)
        vals = jnp.where(fiota == im, jnp.float32("inf"), vals)
    return jnp.concatenate(cols, axis=1).astype(jnp.int32)


def _tc_knn(centers_pad, points_t):
    mt = 128
    _J = 4
    _V = _N // 128

    def body(c_ref, p_ref, o_ref, v_ref):
        c = c_ref[0]
        pt = p_ref[0]
        a_sq = jnp.sum(c * c, axis=1, keepdims=True)
        b_sq = jnp.sum(pt * pt, axis=0, keepdims=True)
        inner = jnp.dot(c, pt, preferred_element_type=jnp.float32)
        d2 = jnp.maximum(a_sq + b_sq - 2.0 * inner, 0.0)
        dist = jnp.sqrt(d2 + 1e-6)
        vals = jnp.where(dist <= _RADIUS, dist, jnp.float32(1e9))
        v_ref[...] = vals

        lane = lax.broadcasted_iota(jnp.int32, (8, 128), 1).astype(jnp.float32)
        ndeep = _J + 1
        groups = []
        for g in range(mt // 8):
            def step(v, carry):
                ms, is_ = carry[:ndeep], carry[ndeep:]
                x = v_ref[pl.ds(g * 8, 8), pl.ds(v * 128, 128)]
                ix = lane + (v * 128).astype(jnp.float32)
                ms, is_ = list(ms), list(is_)
                for j in range(ndeep):
                    cmask = x < ms[j]
                    nm = jnp.where(cmask, x, ms[j])
                    x = jnp.where(cmask, ms[j], x)
                    ni = jnp.where(cmask, ix, is_[j])
                    ix = jnp.where(cmask, is_[j], ix)
                    ms[j], is_[j] = nm, ni
                return tuple(ms) + tuple(is_)

            big = jnp.full((8, 128), 3e9, jnp.float32)
            bigi = jnp.full((8, 128), jnp.float32(_N))
            init = tuple(big for _ in range(ndeep)) + tuple(
                bigi for _ in range(ndeep))
            groups.append(lax.fori_loop(0, _V, step, init))

        cand_v = jnp.concatenate(
            [jnp.concatenate([gr[j] for gr in groups], axis=0)
             for j in range(_J)], axis=1)
        cand_i = jnp.concatenate(
            [jnp.concatenate([gr[ndeep + j] for gr in groups], axis=0)
             for j in range(_J)], axis=1)
        rem = jnp.concatenate([gr[_J] for gr in groups], axis=0)

        cols = []
        m = None
        cv = cand_v
        for _ in range(_K):
            m = jnp.min(cv, axis=1, keepdims=True)
            im = jnp.min(jnp.where(cv == m, cand_i, jnp.float32(_N)),
                         axis=1, keepdims=True)
            cols.append(im)
            cv = jnp.where(cand_i == im, jnp.float32("inf"), cv)
        fast = jnp.concatenate(cols, axis=1).astype(jnp.int32)

        minrem = jnp.min(rem, axis=1, keepdims=True)
        ok = jnp.all(minrem > m)

        @pl.when(ok)
        def _():
            o_ref[0] = fast

        @pl.when(jnp.logical_not(ok))
        def _():
            fiota = lax.broadcasted_iota(
                jnp.int32, (mt, _N), 1).astype(jnp.float32)
            o_ref[0] = _full_extract(v_ref[...], fiota)

    return pl.pallas_call(
        body,
        grid=(_B, _M // mt),
        in_specs=[
            pl.BlockSpec((1, mt, _PD), lambda b, i: (b, i, 0)),
            pl.BlockSpec((1, _PD, _N), lambda b, i: (b, 0, 0)),
        ],
        out_specs=pl.BlockSpec((1, mt, _K), lambda b, i: (b, i, 0)),
        out_shape=jax.ShapeDtypeStruct((_B, _M, _K), jnp.int32),
        scratch_shapes=[pltpu.VMEM((mt, _N), jnp.float32)],
    )(centers_pad, points_t)


def _tc_mlp(neigh_rows, cf, ln1_g, ln1_b, W1, b1, W2, b2, Wm, bm, res_scale,
            ln2_g, ln2_b):
    mt = 128
    rows = mt * _K

    def body(n_ref, cf_ref, g1_ref, bb1_ref, w1_ref, b1_ref, w2_ref, b2_ref,
             wm_ref, bm_ref, rs_ref, g2_ref, bb2_ref, o_ref):
        nb = n_ref[0]
        cfb = cf_ref[0]
        cfe = jnp.reshape(
            jnp.broadcast_to(cfb[:, None, :], (mt, _K, _C)), (rows, _C))
        combo = jnp.concatenate([cfe, nb], axis=1)
        mu = jnp.mean(combo, axis=1, keepdims=True)
        var = jnp.mean((combo - mu) ** 2, axis=1, keepdims=True)
        cn = (combo - mu) / jnp.sqrt(var + 1e-5) * g1_ref[0] + bb1_ref[0]
        h = jnp.maximum(
            jnp.dot(cn, w1_ref[...], preferred_element_type=jnp.float32)
            + b1_ref[0], 0.0)
        cw = jax.nn.sigmoid(
            jnp.dot(h, w2_ref[...], preferred_element_type=jnp.float32)
            + b2_ref[0])
        w = nb * cw
        wm = jnp.mean(jnp.reshape(w, (mt, _K, _C)), axis=1)
        fused = cfb + wm * rs_ref[0, 0]
        o = jnp.maximum(
            jnp.dot(fused, wm_ref[...], preferred_element_type=jnp.float32)
            + bm_ref[0], 0.0)
        mu2 = jnp.mean(o, axis=1, keepdims=True)
        var2 = jnp.mean((o - mu2) ** 2, axis=1, keepdims=True)
        o_ref[0] = (o - mu2) / jnp.sqrt(var2 + 1e-5) * g2_ref[0] + bb2_ref[0]

    def full(shape):
        return pl.BlockSpec(shape, lambda b, i: tuple(0 for _ in shape))

    return pl.pallas_call(
        body,
        grid=(_B, _M // mt),
        in_specs=[
            pl.BlockSpec((1, rows, _C), lambda b, i: (b, i, 0)),
            pl.BlockSpec((1, mt, _C), lambda b, i: (b, i, 0)),
            full((1, 2 * _C)),
            full((1, 2 * _C)),
            full((2 * _C, _C)),
            full((1, _C)),
            full((_C, _C)),
            full((1, _C)),
            full((_C, _O)),
            full((1, _O)),
            full((1, 1)),
            full((1, _O)),
            full((1, _O)),
        ],
        out_specs=pl.BlockSpec((1, mt, _O), lambda b, i: (b, i, 0)),
        out_shape=jax.ShapeDtypeStruct((_B, _M, _O), jnp.float32),
    )(neigh_rows, cf, ln1_g.reshape(1, -1), ln1_b.reshape(1, -1), W1,
      b1.reshape(1, -1), W2, b2.reshape(1, -1), Wm, bm.reshape(1, -1),
      res_scale.reshape(1, 1), ln2_g.reshape(1, -1), ln2_b.reshape(1, -1))


def kernel(points, feats, center_idx, ln1_g, ln1_b, W1, b1, W2, b2, Wm, bm,
           res_scale, ln2_g, ln2_b):
    points_pad = jnp.pad(points, ((0, 0), (0, 0), (0, _PD - 3)))
    points_pad128 = jnp.pad(points, ((0, 0), (0, 0), (0, 128 - 3)))
    points_pad2d = points_pad128.reshape(_B * _N, 128)
    feats2d = feats.reshape(_B * _N, _C)
    ci_flat = center_idx.reshape(_B * _M)

    centers_rows = _gather_rows_sc(points_pad2d, ci_flat, 128, _M)
    cf_rows = _gather_rows_sc(feats2d, ci_flat, _C, _M)
    centers_pad = centers_rows.reshape(_B, _M, 128)[:, :, :_PD]
    cf = cf_rows.reshape(_B, _M, _C)

    points_t = jnp.transpose(points_pad, (0, 2, 1))
    knn_idx = _tc_knn(centers_pad, points_t)

    knn_flat = knn_idx.reshape(_B * _M * _K)
    neigh_rows = _gather_rows_sc(feats2d, knn_flat, _C, _M * _K)
    neigh = neigh_rows.reshape(_B, _M * _K, _C)

    out = _tc_mlp(neigh, cf, ln1_g, ln1_b, W1, b1, W2, b2, Wm, bm,
                  res_scale, ln2_g, ln2_b)
    return out, knn_idx


# speedup vs baseline: 1.5807x; 1.5807x over previous
import functools

import jax
import jax.numpy as jnp
from jax import lax
from jax.experimental import pallas as pl
from jax.experimental.pallas import tpu as pltpu
from jax.experimental.pallas import tpu_sc as plsc

_B, _N, _M, _C, _O, _K = 8, 4096, 1024, 128, 128, 16
_RADIUS = 10.0
_PD = 16
_NC, _NS = 2, 16
_NW = _NC * _NS
_CH = 128


def _sc_mesh():
    return plsc.VectorSubcoreMesh(
        core_axis_name="c", subcore_axis_name="s",
        num_cores=_NC, num_subcores=_NS)


def _gather_rows_sc(table2d, idx_flat, width, rows_per_batch, batch0=0):
    rows = idx_flat.shape[0]
    rows_per_w = rows // _NW
    nch = rows_per_w // _CH

    @functools.partial(
        pl.kernel,
        out_type=jax.ShapeDtypeStruct((rows, width), jnp.float32),
        mesh=_sc_mesh(),
        scratch_types=[
            pltpu.VMEM((_CH,), jnp.int32),
            pltpu.VMEM((_CH, width), jnp.float32),
            pltpu.SemaphoreType.DMA,
        ],
    )
    def k(tab_hbm, idx_hbm, out_hbm, idx_v, rows_v, sem):
        wid = lax.axis_index("s") * _NC + lax.axis_index("c")
        boff = (batch0 + wid * rows_per_w // rows_per_batch) * _N
        for t in range(nch):
            base = wid * rows_per_w + t * _CH
            pltpu.sync_copy(idx_hbm.at[pl.ds(base, _CH)], idx_v)
            for j in range(_CH // 16):
                sl = pl.ds(j * 16, 16)
                idx_v[sl] = idx_v[sl] + boff
            pltpu.async_copy(tab_hbm.at[idx_v], rows_v, sem).wait()
            pltpu.sync_copy(rows_v, out_hbm.at[pl.ds(base, _CH)])

    return k(table2d, idx_flat)


def _full_extract(vals, fiota):
    cols = []
    for _ in range(_K):
        m = jnp.min(vals, axis=1, keepdims=True)
        im = jnp.min(jnp.where(vals == m, fiota, jnp.float32(_N)),
                     axis=1, keepdims=True)
        cols.append(im)
        vals = jnp.where(fiota == im, jnp.float32("inf"), vals)
    return jnp.concatenate(cols, axis=1).astype(jnp.int32)


def _tc_knn(centers_pad, points_t):
    mt = 128
    _J = 4
    _V = _N // 128

    def body(c_ref, p_ref, o_ref, v_ref):
        c = c_ref[0]
        pt = p_ref[0]
        a_sq = jnp.sum(c * c, axis=1, keepdims=True)
        b_sq = jnp.sum(pt * pt, axis=0, keepdims=True)
        inner = jnp.dot(c, pt, preferred_element_type=jnp.float32)
        d2 = jnp.maximum(a_sq + b_sq - 2.0 * inner, 0.0)
        dist = jnp.sqrt(d2 + 1e-6)
        vals = jnp.where(dist <= _RADIUS, dist, jnp.float32(1e9))
        v_ref[...] = vals

        lane = lax.broadcasted_iota(jnp.int32, (8, 128), 1).astype(jnp.float32)
        ndeep = _J + 1
        groups = []
        for g in range(mt // 8):
            big = jnp.full((8, 128), 3e9, jnp.float32)
            bigi = jnp.full((8, 128), jnp.float32(_N))
            ms = [big] * ndeep
            is_ = [bigi] * ndeep
            vg = vals[g * 8:(g + 1) * 8, :]
            for v in range(_V):
                x = vg[:, v * 128:(v + 1) * 128]
                ix = lane + jnp.float32(v * 128)
                for j in range(ndeep):
                    cmask = x < ms[j]
                    nm = jnp.where(cmask, x, ms[j])
                    x = jnp.where(cmask, ms[j], x)
                    ni = jnp.where(cmask, ix, is_[j])
                    ix = jnp.where(cmask, is_[j], ix)
                    ms[j], is_[j] = nm, ni
            groups.append(tuple(ms) + tuple(is_))

        cand_v = jnp.concatenate(
            [jnp.concatenate([gr[j] for gr in groups], axis=0)
             for j in range(_J)], axis=1)
        cand_i = jnp.concatenate(
            [jnp.concatenate([gr[ndeep + j] for gr in groups], axis=0)
             for j in range(_J)], axis=1)
        rem = jnp.concatenate([gr[_J] for gr in groups], axis=0)

        cols = []
        m = None
        cv = cand_v
        for _ in range(_K):
            m = jnp.min(cv, axis=1, keepdims=True)
            im = jnp.min(jnp.where(cv == m, cand_i, jnp.float32(_N)),
                         axis=1, keepdims=True)
            cols.append(im)
            cv = jnp.where(cand_i == im, jnp.float32("inf"), cv)
        fast = jnp.concatenate(cols, axis=1).astype(jnp.int32)

        minrem = jnp.min(rem, axis=1, keepdims=True)
        ok = jnp.all(minrem > m)

        @pl.when(ok)
        def _():
            o_ref[0] = fast

        @pl.when(jnp.logical_not(ok))
        def _():
            fiota = lax.broadcasted_iota(
                jnp.int32, (mt, _N), 1).astype(jnp.float32)
            o_ref[0] = _full_extract(v_ref[...], fiota)

    nb = centers_pad.shape[0]
    return pl.pallas_call(
        body,
        grid=(nb, _M // mt),
        in_specs=[
            pl.BlockSpec((1, mt, _PD), lambda b, i: (b, i, 0)),
            pl.BlockSpec((1, _PD, _N), lambda b, i: (b, 0, 0)),
        ],
        out_specs=pl.BlockSpec((1, mt, _K), lambda b, i: (b, i, 0)),
        out_shape=jax.ShapeDtypeStruct((nb, _M, _K), jnp.int32),
        scratch_shapes=[pltpu.VMEM((mt, _N), jnp.float32)],
    )(centers_pad, points_t)


def _tc_mlp(neigh_rows, cf, ln1_g, ln1_b, W1, b1, W2, b2, Wm, bm, res_scale,
            ln2_g, ln2_b):
    mt = 128
    rows = mt * _K

    def body(n_ref, cf_ref, g1_ref, bb1_ref, w1_ref, b1_ref, w2_ref, b2_ref,
             wm_ref, bm_ref, rs_ref, g2_ref, bb2_ref, o_ref):
        nb = n_ref[0]
        cfb = cf_ref[0]
        g1 = g1_ref[0]
        w1 = w1_ref[...]

        def expand(x):
            return jnp.reshape(
                jnp.broadcast_to(x[:, None, :], (mt, _K, x.shape[1])),
                (rows, x.shape[1]))

        s_c = jnp.sum(cfb, axis=1, keepdims=True)
        q_c = jnp.sum(cfb * cfb, axis=1, keepdims=True)
        s_n = jnp.sum(nb, axis=1, keepdims=True)
        q_n = jnp.sum(nb * nb, axis=1, keepdims=True)
        mu = (expand(s_c) + s_n) * jnp.float32(1.0 / (2 * _C))
        var = (expand(q_c) + q_n) * jnp.float32(1.0 / (2 * _C)) - mu * mu
        inv = lax.rsqrt(var + 1e-5)

        t1 = jnp.dot(cfb * g1[None, :_C], w1[:_C, :],
                     preferred_element_type=jnp.float32)
        t2 = jnp.dot(nb * g1[None, _C:], w1[_C:, :],
                     preferred_element_type=jnp.float32)
        gw1 = jnp.dot(g1[None, :], w1,
                      preferred_element_type=jnp.float32)
        bw1 = jnp.dot(bb1_ref[0][None, :], w1,
                      preferred_element_type=jnp.float32)
        h = jnp.maximum(
            inv * (expand(t1) + t2 - mu * gw1) + (bw1 + b1_ref[0]), 0.0)
        cw = jax.nn.sigmoid(
            jnp.dot(h, w2_ref[...], preferred_element_type=jnp.float32)
            + b2_ref[0])
        w = nb * cw
        wm = jnp.mean(jnp.reshape(w, (mt, _K, _C)), axis=1)
        fused = cfb + wm * rs_ref[0, 0]
        o = jnp.maximum(
            jnp.dot(fused, wm_ref[...], preferred_element_type=jnp.float32)
            + bm_ref[0], 0.0)
        mu2 = jnp.mean(o, axis=1, keepdims=True)
        var2 = jnp.mean((o - mu2) ** 2, axis=1, keepdims=True)
        o_ref[0] = (o - mu2) / jnp.sqrt(var2 + 1e-5) * g2_ref[0] + bb2_ref[0]

    def full(shape):
        return pl.BlockSpec(shape, lambda b, i: tuple(0 for _ in shape))

    nb = cf.shape[0]
    return pl.pallas_call(
        body,
        grid=(nb, _M // mt),
        in_specs=[
            pl.BlockSpec((1, rows, _C), lambda b, i: (b, i, 0)),
            pl.BlockSpec((1, mt, _C), lambda b, i: (b, i, 0)),
            full((1, 2 * _C)),
            full((1, 2 * _C)),
            full((2 * _C, _C)),
            full((1, _C)),
            full((_C, _C)),
            full((1, _C)),
            full((_C, _O)),
            full((1, _O)),
            full((1, 1)),
            full((1, _O)),
            full((1, _O)),
        ],
        out_specs=pl.BlockSpec((1, mt, _O), lambda b, i: (b, i, 0)),
        out_shape=jax.ShapeDtypeStruct((nb, _M, _O), jnp.float32),
    )(neigh_rows, cf, ln1_g.reshape(1, -1), ln1_b.reshape(1, -1), W1,
      b1.reshape(1, -1), W2, b2.reshape(1, -1), Wm, bm.reshape(1, -1),
      res_scale.reshape(1, 1), ln2_g.reshape(1, -1), ln2_b.reshape(1, -1))


def kernel(points, feats, center_idx, ln1_g, ln1_b, W1, b1, W2, b2, Wm, bm,
           res_scale, ln2_g, ln2_b):
    points_pad = jnp.pad(points, ((0, 0), (0, 0), (0, _PD - 3)))
    points_pad128 = jnp.pad(points, ((0, 0), (0, 0), (0, 128 - 3)))
    points_pad2d = points_pad128.reshape(_B * _N, 128)
    feats2d = feats.reshape(_B * _N, _C)
    ci_flat = center_idx.reshape(_B * _M)

    centers_rows = _gather_rows_sc(points_pad2d, ci_flat, 128, _M)
    cf_rows = _gather_rows_sc(feats2d, ci_flat, _C, _M)
    centers_pad = centers_rows.reshape(_B, _M, 128)[:, :, :_PD]
    cf = cf_rows.reshape(_B, _M, _C)
    points_t = jnp.transpose(points_pad, (0, 2, 1))

    hb = _B // 2
    outs, knns = [], []
    for h in range(2):
        sl = slice(h * hb, (h + 1) * hb)
        knn_idx = _tc_knn(centers_pad[sl], points_t[sl])
        knn_flat = knn_idx.reshape(hb * _M * _K)
        neigh_rows = _gather_rows_sc(feats2d, knn_flat, _C, _M * _K,
                                     batch0=h * hb)
        neigh = neigh_rows.reshape(hb, _M * _K, _C)
        out_h = _tc_mlp(neigh, cf[sl], ln1_g, ln1_b, W1, b1, W2, b2, Wm, bm,
                        res_scale, ln2_g, ln2_b)
        outs.append(out_h)
        knns.append(knn_idx)
    return (jnp.concatenate(outs, axis=0), jnp.concatenate(knns, axis=0))


# speedup vs baseline: 1.7028x; 1.0772x over previous
import functools

import jax
import jax.numpy as jnp
from jax import lax
from jax.experimental import pallas as pl
from jax.experimental.pallas import tpu as pltpu
from jax.experimental.pallas import tpu_sc as plsc

_B, _N, _M, _C, _O, _K = 8, 4096, 1024, 128, 128, 16
_RADIUS = 10.0
_PD = 16
_NC, _NS = 2, 16
_NW = _NC * _NS
_CH = 128


def _sc_mesh():
    return plsc.VectorSubcoreMesh(
        core_axis_name="c", subcore_axis_name="s",
        num_cores=_NC, num_subcores=_NS)


def _gather_rows_sc(table2d, idx_flat, width, rows_per_batch, batch0=0):
    rows = idx_flat.shape[0]
    rows_per_w = rows // _NW
    nch = rows_per_w // _CH

    @functools.partial(
        pl.kernel,
        out_type=jax.ShapeDtypeStruct((rows, width), jnp.float32),
        mesh=_sc_mesh(),
        scratch_types=[
            pltpu.VMEM((_CH,), jnp.int32),
            pltpu.VMEM((_CH, width), jnp.float32),
            pltpu.SemaphoreType.DMA,
        ],
    )
    def k(tab_hbm, idx_hbm, out_hbm, idx_v, rows_v, sem):
        wid = lax.axis_index("s") * _NC + lax.axis_index("c")
        boff = (batch0 + wid * rows_per_w // rows_per_batch) * _N
        for t in range(nch):
            base = wid * rows_per_w + t * _CH
            pltpu.sync_copy(idx_hbm.at[pl.ds(base, _CH)], idx_v)
            for j in range(_CH // 16):
                sl = pl.ds(j * 16, 16)
                idx_v[sl] = idx_v[sl] + boff
            pltpu.async_copy(tab_hbm.at[idx_v], rows_v, sem).wait()
            pltpu.sync_copy(rows_v, out_hbm.at[pl.ds(base, _CH)])

    return k(table2d, idx_flat)


def _full_extract(vals, fiota):
    cols = []
    for _ in range(_K):
        m = jnp.min(vals, axis=1, keepdims=True)
        im = jnp.min(jnp.where(vals == m, fiota, jnp.float32(_N)),
                     axis=1, keepdims=True)
        cols.append(im)
        vals = jnp.where(fiota == im, jnp.float32("inf"), vals)
    return jnp.concatenate(cols, axis=1).astype(jnp.int32)


def _tc_knn(centers_pad, points_t):
    mt = 128
    _J = 4
    _V = _N // 128

    def body(c_ref, p_ref, o_ref, v_ref):
        c = c_ref[0]
        pt = p_ref[0]
        a_sq = jnp.sum(c * c, axis=1, keepdims=True)
        b_sq = jnp.sum(pt * pt, axis=0, keepdims=True)
        inner = jnp.dot(c, pt, preferred_element_type=jnp.float32)
        d2 = a_sq + b_sq - 2.0 * inner
        v_ref[...] = d2

        lane = lax.broadcasted_iota(jnp.int32, (8, 128), 1).astype(jnp.float32)
        ndeep = _J + 1
        groups = []
        for g in range(mt // 8):
            big = jnp.full((8, 128), 3e9, jnp.float32)
            bigi = jnp.full((8, 128), jnp.float32(_N))
            ms = [big] * ndeep
            is_ = [bigi] * ndeep
            vg = d2[g * 8:(g + 1) * 8, :]
            for v in range(_V):
                x = vg[:, v * 128:(v + 1) * 128]
                ix = lane + jnp.float32(v * 128)
                for j in range(ndeep):
                    cmask = x < ms[j]
                    nm = jnp.where(cmask, x, ms[j])
                    x = jnp.where(cmask, ms[j], x)
                    ni = jnp.where(cmask, ix, is_[j])
                    ix = jnp.where(cmask, is_[j], ix)
                    ms[j], is_[j] = nm, ni
            groups.append(tuple(ms) + tuple(is_))

        def masked_dist(d2x):
            s = jnp.sqrt(jnp.maximum(d2x, 0.0) + 1e-6)
            return jnp.where(s <= _RADIUS, s, jnp.float32(1e9))

        cand_v = masked_dist(jnp.concatenate(
            [jnp.concatenate([gr[j] for gr in groups], axis=0)
             for j in range(_J)], axis=1))
        cand_i = jnp.concatenate(
            [jnp.concatenate([gr[ndeep + j] for gr in groups], axis=0)
             for j in range(_J)], axis=1)
        rem = masked_dist(
            jnp.concatenate([gr[_J] for gr in groups], axis=0))

        cols = []
        m = None
        cv = cand_v
        for _ in range(_K):
            m = jnp.min(cv, axis=1, keepdims=True)
            im = jnp.min(jnp.where(cv == m, cand_i, jnp.float32(_N)),
                         axis=1, keepdims=True)
            cols.append(im)
            cv = jnp.where(cand_i == im, jnp.float32("inf"), cv)
        fast = jnp.concatenate(cols, axis=1).astype(jnp.int32)

        minrem = jnp.min(rem, axis=1, keepdims=True)
        ok = jnp.all(minrem > m)

        @pl.when(ok)
        def _():
            o_ref[0] = fast

        @pl.when(jnp.logical_not(ok))
        def _():
            fiota = lax.broadcasted_iota(
                jnp.int32, (mt, _N), 1).astype(jnp.float32)
            o_ref[0] = _full_extract(masked_dist(v_ref[...]), fiota)

    nb = centers_pad.shape[0]
    return pl.pallas_call(
        body,
        grid=(nb, _M // mt),
        in_specs=[
            pl.BlockSpec((1, mt, _PD), lambda b, i: (b, i, 0)),
            pl.BlockSpec((1, _PD, _N), lambda b, i: (b, 0, 0)),
        ],
        out_specs=pl.BlockSpec((1, mt, _K), lambda b, i: (b, i, 0)),
        out_shape=jax.ShapeDtypeStruct((nb, _M, _K), jnp.int32),
        scratch_shapes=[pltpu.VMEM((mt, _N), jnp.float32)],
    )(centers_pad, points_t)


def _tc_mlp(neigh_rows, cf, ln1_g, ln1_b, W1, b1, W2, b2, Wm, bm, res_scale,
            ln2_g, ln2_b):
    mt = 128
    rows = mt * _K

    def body(n_ref, cf_ref, g1_ref, bb1_ref, w1_ref, b1_ref, w2_ref, b2_ref,
             wm_ref, bm_ref, rs_ref, g2_ref, bb2_ref, o_ref):
        nb = n_ref[0]
        cfb = cf_ref[0]
        g1 = g1_ref[0]
        w1 = w1_ref[...]

        def expand(x):
            return jnp.reshape(
                jnp.broadcast_to(x[:, None, :], (mt, _K, x.shape[1])),
                (rows, x.shape[1]))

        s_c = jnp.sum(cfb, axis=1, keepdims=True)
        q_c = jnp.sum(cfb * cfb, axis=1, keepdims=True)
        s_n = jnp.sum(nb, axis=1, keepdims=True)
        q_n = jnp.sum(nb * nb, axis=1, keepdims=True)
        mu = (expand(s_c) + s_n) * jnp.float32(1.0 / (2 * _C))
        var = (expand(q_c) + q_n) * jnp.float32(1.0 / (2 * _C)) - mu * mu
        inv = lax.rsqrt(var + 1e-5)

        t1 = jnp.dot(cfb * g1[None, :_C], w1[:_C, :],
                     preferred_element_type=jnp.float32)
        t2 = jnp.dot(nb * g1[None, _C:], w1[_C:, :],
                     preferred_element_type=jnp.float32)
        gw1 = jnp.dot(g1[None, :], w1,
                      preferred_element_type=jnp.float32)
        bw1 = jnp.dot(bb1_ref[0][None, :], w1,
                      preferred_element_type=jnp.float32)
        h = jnp.maximum(
            inv * (expand(t1) + t2 - mu * gw1) + (bw1 + b1_ref[0]), 0.0)
        cw = jax.nn.sigmoid(
            jnp.dot(h, w2_ref[...], preferred_element_type=jnp.float32)
            + b2_ref[0])
        w = nb * cw
        wm = jnp.mean(jnp.reshape(w, (mt, _K, _C)), axis=1)
        fused = cfb + wm * rs_ref[0, 0]
        o = jnp.maximum(
            jnp.dot(fused, wm_ref[...], preferred_element_type=jnp.float32)
            + bm_ref[0], 0.0)
        mu2 = jnp.mean(o, axis=1, keepdims=True)
        var2 = jnp.mean((o - mu2) ** 2, axis=1, keepdims=True)
        o_ref[0] = (o - mu2) / jnp.sqrt(var2 + 1e-5) * g2_ref[0] + bb2_ref[0]

    def full(shape):
        return pl.BlockSpec(shape, lambda b, i: tuple(0 for _ in shape))

    nb = cf.shape[0]
    return pl.pallas_call(
        body,
        grid=(nb, _M // mt),
        in_specs=[
            pl.BlockSpec((1, rows, _C), lambda b, i: (b, i, 0)),
            pl.BlockSpec((1, mt, _C), lambda b, i: (b, i, 0)),
            full((1, 2 * _C)),
            full((1, 2 * _C)),
            full((2 * _C, _C)),
            full((1, _C)),
            full((_C, _C)),
            full((1, _C)),
            full((_C, _O)),
            full((1, _O)),
            full((1, 1)),
            full((1, _O)),
            full((1, _O)),
        ],
        out_specs=pl.BlockSpec((1, mt, _O), lambda b, i: (b, i, 0)),
        out_shape=jax.ShapeDtypeStruct((nb, _M, _O), jnp.float32),
    )(neigh_rows, cf, ln1_g.reshape(1, -1), ln1_b.reshape(1, -1), W1,
      b1.reshape(1, -1), W2, b2.reshape(1, -1), Wm, bm.reshape(1, -1),
      res_scale.reshape(1, 1), ln2_g.reshape(1, -1), ln2_b.reshape(1, -1))


def kernel(points, feats, center_idx, ln1_g, ln1_b, W1, b1, W2, b2, Wm, bm,
           res_scale, ln2_g, ln2_b):
    points_pad = jnp.pad(points, ((0, 0), (0, 0), (0, _PD - 3)))
    points_pad128 = jnp.pad(points, ((0, 0), (0, 0), (0, 128 - 3)))
    points_pad2d = points_pad128.reshape(_B * _N, 128)
    feats2d = feats.reshape(_B * _N, _C)
    ci_flat = center_idx.reshape(_B * _M)

    centers_rows = _gather_rows_sc(points_pad2d, ci_flat, 128, _M)
    cf_rows = _gather_rows_sc(feats2d, ci_flat, _C, _M)
    centers_pad = centers_rows.reshape(_B, _M, 128)[:, :, :_PD]
    cf = cf_rows.reshape(_B, _M, _C)
    points_t = jnp.transpose(points_pad, (0, 2, 1))

    hb = _B // 2
    outs, knns = [], []
    for h in range(2):
        sl = slice(h * hb, (h + 1) * hb)
        knn_idx = _tc_knn(centers_pad[sl], points_t[sl])
        knn_flat = knn_idx.reshape(hb * _M * _K)
        neigh_rows = _gather_rows_sc(feats2d, knn_flat, _C, _M * _K,
                                     batch0=h * hb)
        neigh = neigh_rows.reshape(hb, _M * _K, _C)
        out_h = _tc_mlp(neigh, cf[sl], ln1_g, ln1_b, W1, b1, W2, b2, Wm, bm,
                        res_scale, ln2_g, ln2_b)
        outs.append(out_h)
        knns.append(knn_idx)
    return (jnp.concatenate(outs, axis=0), jnp.concatenate(knns, axis=0))


# speedup vs baseline: 1.9894x; 1.1683x over previous
import functools

import jax
import jax.numpy as jnp
from jax import lax
from jax.experimental import pallas as pl
from jax.experimental.pallas import tpu as pltpu
from jax.experimental.pallas import tpu_sc as plsc

_B, _N, _M, _C, _O, _K = 8, 4096, 1024, 128, 128, 16
_RADIUS = 10.0
_PD = 16
_NC, _NS = 2, 16
_NW = _NC * _NS
_CH = 128


def _sc_mesh():
    return plsc.VectorSubcoreMesh(
        core_axis_name="c", subcore_axis_name="s",
        num_cores=_NC, num_subcores=_NS)


def _gather_rows_sc(table2d, idx_flat, width, rows_per_batch, batch0=0):
    rows = idx_flat.shape[0]
    rows_per_w = rows // _NW
    nch = rows_per_w // _CH

    @functools.partial(
        pl.kernel,
        out_type=jax.ShapeDtypeStruct((rows, width), jnp.float32),
        mesh=_sc_mesh(),
        scratch_types=[
            pltpu.VMEM((_CH,), jnp.int32),
            pltpu.VMEM((_CH, width), jnp.float32),
            pltpu.SemaphoreType.DMA,
        ],
    )
    def k(tab_hbm, idx_hbm, out_hbm, idx_v, rows_v, sem):
        wid = lax.axis_index("s") * _NC + lax.axis_index("c")
        boff = (batch0 + wid * rows_per_w // rows_per_batch) * _N
        for t in range(nch):
            base = wid * rows_per_w + t * _CH
            pltpu.sync_copy(idx_hbm.at[pl.ds(base, _CH)], idx_v)
            for j in range(_CH // 16):
                sl = pl.ds(j * 16, 16)
                idx_v[sl] = idx_v[sl] + boff
            pltpu.async_copy(tab_hbm.at[idx_v], rows_v, sem).wait()
            pltpu.sync_copy(rows_v, out_hbm.at[pl.ds(base, _CH)])

    return k(table2d, idx_flat)


def _full_extract(vals, fiota):
    cols = []
    for _ in range(_K):
        m = jnp.min(vals, axis=1, keepdims=True)
        im = jnp.min(jnp.where(vals == m, fiota, jnp.float32(_N)),
                     axis=1, keepdims=True)
        cols.append(im)
        vals = jnp.where(fiota == im, jnp.float32("inf"), vals)
    return jnp.concatenate(cols, axis=1).astype(jnp.int32)


def _tc_knn(centers_pad, points_t):
    mt = 256
    _J = 4
    _V = _N // 128

    def body(c_ref, p_ref, o_ref, v_ref):
        c = c_ref[0]
        pt = p_ref[0]
        a_sq = jnp.sum(c * c, axis=1, keepdims=True)
        b_sq = jnp.sum(pt * pt, axis=0, keepdims=True)
        inner = jnp.dot(c, pt, preferred_element_type=jnp.float32)
        d2 = a_sq + b_sq - 2.0 * inner
        v_ref[...] = d2

        lane = lax.broadcasted_iota(jnp.int32, (8, 128), 1).astype(jnp.float32)
        ndeep = _J + 1
        groups = []
        for g in range(mt // 8):
            big = jnp.full((8, 128), 3e9, jnp.float32)
            bigi = jnp.full((8, 128), jnp.float32(_N))
            ms = [big] * ndeep
            is_ = [bigi] * ndeep
            vg = d2[g * 8:(g + 1) * 8, :]
            for v in range(_V):
                x = vg[:, v * 128:(v + 1) * 128]
                ix = lane + jnp.float32(v * 128)
                for j in range(ndeep):
                    cmask = x < ms[j]
                    nm = jnp.where(cmask, x, ms[j])
                    x = jnp.where(cmask, ms[j], x)
                    ni = jnp.where(cmask, ix, is_[j])
                    ix = jnp.where(cmask, is_[j], ix)
                    ms[j], is_[j] = nm, ni
            groups.append(tuple(ms) + tuple(is_))

        def masked_dist(d2x):
            s = jnp.sqrt(jnp.maximum(d2x, 0.0) + 1e-6)
            return jnp.where(s <= _RADIUS, s, jnp.float32(1e9))

        cand_v = masked_dist(jnp.concatenate(
            [jnp.concatenate([gr[j] for gr in groups], axis=0)
             for j in range(_J)], axis=1))
        cand_i = jnp.concatenate(
            [jnp.concatenate([gr[ndeep + j] for gr in groups], axis=0)
             for j in range(_J)], axis=1)
        rem = masked_dist(
            jnp.concatenate([gr[_J] for gr in groups], axis=0))

        cols = []
        m = None
        cv = cand_v
        for _ in range(_K):
            m = jnp.min(cv, axis=1, keepdims=True)
            im = jnp.min(jnp.where(cv == m, cand_i, jnp.float32(_N)),
                         axis=1, keepdims=True)
            cols.append(im)
            cv = jnp.where(cand_i == im, jnp.float32("inf"), cv)
        fast = jnp.concatenate(cols, axis=1).astype(jnp.int32)

        minrem = jnp.min(rem, axis=1, keepdims=True)
        ok = jnp.all(minrem > m)

        @pl.when(ok)
        def _():
            o_ref[0] = fast

        @pl.when(jnp.logical_not(ok))
        def _():
            fiota = lax.broadcasted_iota(
                jnp.int32, (mt, _N), 1).astype(jnp.float32)
            o_ref[0] = _full_extract(masked_dist(v_ref[...]), fiota)

    nb = centers_pad.shape[0]
    return pl.pallas_call(
        body,
        grid=(nb, _M // mt),
        in_specs=[
            pl.BlockSpec((1, mt, _PD), lambda b, i: (b, i, 0)),
            pl.BlockSpec((1, _PD, _N), lambda b, i: (b, 0, 0)),
        ],
        out_specs=pl.BlockSpec((1, mt, _K), lambda b, i: (b, i, 0)),
        out_shape=jax.ShapeDtypeStruct((nb, _M, _K), jnp.int32),
        scratch_shapes=[pltpu.VMEM((mt, _N), jnp.float32)],
    )(centers_pad, points_t)


def _tc_mlp(neigh_rows, cf, ln1_g, ln1_b, W1, b1, W2, b2, Wm, bm, res_scale,
            ln2_g, ln2_b):
    mt = 128
    rows = mt * _K

    def body(n_ref, cf_ref, g1_ref, bb1_ref, w1_ref, b1_ref, w2_ref, b2_ref,
             wm_ref, bm_ref, rs_ref, g2_ref, bb2_ref, o_ref):
        nb = n_ref[0]
        cfb = cf_ref[0]
        g1 = g1_ref[0]
        w1 = w1_ref[...]

        def expand(x):
            return jnp.reshape(
                jnp.broadcast_to(x[:, None, :], (mt, _K, x.shape[1])),
                (rows, x.shape[1]))

        s_c = jnp.sum(cfb, axis=1, keepdims=True)
        q_c = jnp.sum(cfb * cfb, axis=1, keepdims=True)
        s_n = jnp.sum(nb, axis=1, keepdims=True)
        q_n = jnp.sum(nb * nb, axis=1, keepdims=True)
        mu = (expand(s_c) + s_n) * jnp.float32(1.0 / (2 * _C))
        var = (expand(q_c) + q_n) * jnp.float32(1.0 / (2 * _C)) - mu * mu
        inv = lax.rsqrt(var + 1e-5)

        t1 = jnp.dot(cfb * g1[None, :_C], w1[:_C, :],
                     preferred_element_type=jnp.float32)
        t2 = jnp.dot(nb * g1[None, _C:], w1[_C:, :],
                     preferred_element_type=jnp.float32)
        gw1 = jnp.dot(g1[None, :], w1,
                      preferred_element_type=jnp.float32)
        bw1 = jnp.dot(bb1_ref[0][None, :], w1,
                      preferred_element_type=jnp.float32)
        h = jnp.maximum(
            inv * (expand(t1) + t2 - mu * gw1) + (bw1 + b1_ref[0]), 0.0)
        cw = jax.nn.sigmoid(
            jnp.dot(h, w2_ref[...], preferred_element_type=jnp.float32)
            + b2_ref[0])
        w = nb * cw
        wm = jnp.mean(jnp.reshape(w, (mt, _K, _C)), axis=1)
        fused = cfb + wm * rs_ref[0, 0]
        o = jnp.maximum(
            jnp.dot(fused, wm_ref[...], preferred_element_type=jnp.float32)
            + bm_ref[0], 0.0)
        mu2 = jnp.mean(o, axis=1, keepdims=True)
        var2 = jnp.mean((o - mu2) ** 2, axis=1, keepdims=True)
        o_ref[0] = (o - mu2) / jnp.sqrt(var2 + 1e-5) * g2_ref[0] + bb2_ref[0]

    def full(shape):
        return pl.BlockSpec(shape, lambda b, i: tuple(0 for _ in shape))

    nb = cf.shape[0]
    return pl.pallas_call(
        body,
        grid=(nb, _M // mt),
        in_specs=[
            pl.BlockSpec((1, rows, _C), lambda b, i: (b, i, 0)),
            pl.BlockSpec((1, mt, _C), lambda b, i: (b, i, 0)),
            full((1, 2 * _C)),
            full((1, 2 * _C)),
            full((2 * _C, _C)),
            full((1, _C)),
            full((_C, _C)),
            full((1, _C)),
            full((_C, _O)),
            full((1, _O)),
            full((1, 1)),
            full((1, _O)),
            full((1, _O)),
        ],
        out_specs=pl.BlockSpec((1, mt, _O), lambda b, i: (b, i, 0)),
        out_shape=jax.ShapeDtypeStruct((nb, _M, _O), jnp.float32),
    )(neigh_rows, cf, ln1_g.reshape(1, -1), ln1_b.reshape(1, -1), W1,
      b1.reshape(1, -1), W2, b2.reshape(1, -1), Wm, bm.reshape(1, -1),
      res_scale.reshape(1, 1), ln2_g.reshape(1, -1), ln2_b.reshape(1, -1))


def kernel(points, feats, center_idx, ln1_g, ln1_b, W1, b1, W2, b2, Wm, bm,
           res_scale, ln2_g, ln2_b):
    points_pad = jnp.pad(points, ((0, 0), (0, 0), (0, _PD - 3)))
    points_pad128 = jnp.pad(points, ((0, 0), (0, 0), (0, 128 - 3)))
    points_pad2d = points_pad128.reshape(_B * _N, 128)
    feats2d = feats.reshape(_B * _N, _C)
    ci_flat = center_idx.reshape(_B * _M)

    centers_rows = _gather_rows_sc(points_pad2d, ci_flat, 128, _M)
    cf_rows = _gather_rows_sc(feats2d, ci_flat, _C, _M)
    centers_pad = centers_rows.reshape(_B, _M, 128)[:, :, :_PD]
    cf = cf_rows.reshape(_B, _M, _C)
    points_t = jnp.transpose(points_pad, (0, 2, 1))

    hb = _B // 2
    outs, knns = [], []
    for h in range(2):
        sl = slice(h * hb, (h + 1) * hb)
        knn_idx = _tc_knn(centers_pad[sl], points_t[sl])
        knn_flat = knn_idx.reshape(hb * _M * _K)
        neigh_rows = _gather_rows_sc(feats2d, knn_flat, _C, _M * _K,
                                     batch0=h * hb)
        neigh = neigh_rows.reshape(hb, _M * _K, _C)
        out_h = _tc_mlp(neigh, cf[sl], ln1_g, ln1_b, W1, b1, W2, b2, Wm, bm,
                        res_scale, ln2_g, ln2_b)
        outs.append(out_h)
        knns.append(knn_idx)
    return (jnp.concatenate(outs, axis=0), jnp.concatenate(knns, axis=0))


# speedup vs baseline: 1.9975x; 1.0041x over previous
import functools

import jax
import jax.numpy as jnp
from jax import lax
from jax.experimental import pallas as pl
from jax.experimental.pallas import tpu as pltpu
from jax.experimental.pallas import tpu_sc as plsc

_B, _N, _M, _C, _O, _K = 8, 4096, 1024, 128, 128, 16
_RADIUS = 10.0
_PD = 16
_NC, _NS = 2, 16
_NW = _NC * _NS
_CH = 128


def _sc_mesh():
    return plsc.VectorSubcoreMesh(
        core_axis_name="c", subcore_axis_name="s",
        num_cores=_NC, num_subcores=_NS)


def _gather_rows_sc(table2d, idx_flat, width, rows_per_batch, batch0=0):
    rows = idx_flat.shape[0]
    rows_per_w = rows // _NW
    nch = rows_per_w // _CH

    @functools.partial(
        pl.kernel,
        out_type=jax.ShapeDtypeStruct((rows, width), jnp.float32),
        mesh=_sc_mesh(),
        scratch_types=[
            pltpu.VMEM((_CH,), jnp.int32),
            pltpu.VMEM((_CH, width), jnp.float32),
            pltpu.SemaphoreType.DMA,
        ],
    )
    def k(tab_hbm, idx_hbm, out_hbm, idx_v, rows_v, sem):
        wid = lax.axis_index("s") * _NC + lax.axis_index("c")
        boff = (batch0 + wid * rows_per_w // rows_per_batch) * _N
        for t in range(nch):
            base = wid * rows_per_w + t * _CH
            pltpu.sync_copy(idx_hbm.at[pl.ds(base, _CH)], idx_v)
            for j in range(_CH // 16):
                sl = pl.ds(j * 16, 16)
                idx_v[sl] = idx_v[sl] + boff
            pltpu.async_copy(tab_hbm.at[idx_v], rows_v, sem).wait()
            pltpu.sync_copy(rows_v, out_hbm.at[pl.ds(base, _CH)])

    return k(table2d, idx_flat)


def _full_extract(vals, fiota):
    cols = []
    for _ in range(_K):
        m = jnp.min(vals, axis=1, keepdims=True)
        im = jnp.min(jnp.where(vals == m, fiota, jnp.float32(_N)),
                     axis=1, keepdims=True)
        cols.append(im)
        vals = jnp.where(fiota == im, jnp.float32("inf"), vals)
    return jnp.concatenate(cols, axis=1).astype(jnp.int32)


def _tc_knn(centers_pad, points_t):
    mt = 512
    _J = 4
    _V = _N // 128

    def body(c_ref, p_ref, o_ref, v_ref):
        c = c_ref[0]
        pt = p_ref[0]
        a_sq = jnp.sum(c * c, axis=1, keepdims=True)
        b_sq = jnp.sum(pt * pt, axis=0, keepdims=True)
        inner = jnp.dot(c, pt, preferred_element_type=jnp.float32)
        d2 = a_sq + b_sq - 2.0 * inner
        v_ref[...] = d2

        lane = lax.broadcasted_iota(jnp.int32, (8, 128), 1).astype(jnp.float32)
        ndeep = _J + 1
        groups = []
        for g in range(mt // 8):
            big = jnp.full((8, 128), 3e9, jnp.float32)
            bigi = jnp.full((8, 128), jnp.float32(_N))
            ms = [big] * ndeep
            is_ = [bigi] * ndeep
            vg = d2[g * 8:(g + 1) * 8, :]
            for v in range(_V):
                x = vg[:, v * 128:(v + 1) * 128]
                ix = lane + jnp.float32(v * 128)
                for j in range(ndeep):
                    cmask = x < ms[j]
                    nm = jnp.where(cmask, x, ms[j])
                    x = jnp.where(cmask, ms[j], x)
                    ni = jnp.where(cmask, ix, is_[j])
                    ix = jnp.where(cmask, is_[j], ix)
                    ms[j], is_[j] = nm, ni
            groups.append(tuple(ms) + tuple(is_))

        def masked_dist(d2x):
            s = jnp.sqrt(jnp.maximum(d2x, 0.0) + 1e-6)
            return jnp.where(s <= _RADIUS, s, jnp.float32(1e9))

        cand_v = masked_dist(jnp.concatenate(
            [jnp.concatenate([gr[j] for gr in groups], axis=0)
             for j in range(_J)], axis=1))
        cand_i = jnp.concatenate(
            [jnp.concatenate([gr[ndeep + j] for gr in groups], axis=0)
             for j in range(_J)], axis=1)
        rem = masked_dist(
            jnp.concatenate([gr[_J] for gr in groups], axis=0))

        cols = []
        m = None
        cv = cand_v
        for _ in range(_K):
            m = jnp.min(cv, axis=1, keepdims=True)
            im = jnp.min(jnp.where(cv == m, cand_i, jnp.float32(_N)),
                         axis=1, keepdims=True)
            cols.append(im)
            cv = jnp.where(cand_i == im, jnp.float32("inf"), cv)
        fast = jnp.concatenate(cols, axis=1).astype(jnp.int32)

        minrem = jnp.min(rem, axis=1, keepdims=True)
        ok = jnp.all(minrem > m)

        @pl.when(ok)
        def _():
            o_ref[0] = fast

        @pl.when(jnp.logical_not(ok))
        def _():
            fiota = lax.broadcasted_iota(
                jnp.int32, (mt, _N), 1).astype(jnp.float32)
            o_ref[0] = _full_extract(masked_dist(v_ref[...]), fiota)

    nb = centers_pad.shape[0]
    return pl.pallas_call(
        body,
        grid=(nb, _M // mt),
        in_specs=[
            pl.BlockSpec((1, mt, _PD), lambda b, i: (b, i, 0)),
            pl.BlockSpec((1, _PD, _N), lambda b, i: (b, 0, 0)),
        ],
        out_specs=pl.BlockSpec((1, mt, _K), lambda b, i: (b, i, 0)),
        out_shape=jax.ShapeDtypeStruct((nb, _M, _K), jnp.int32),
        scratch_shapes=[pltpu.VMEM((mt, _N), jnp.float32)],
    )(centers_pad, points_t)


def _tc_mlp(neigh_rows, cf, ln1_g, ln1_b, W1, b1, W2, b2, Wm, bm, res_scale,
            ln2_g, ln2_b):
    mt = 128
    rows = mt * _K

    def body(n_ref, cf_ref, g1_ref, bb1_ref, w1_ref, b1_ref, w2_ref, b2_ref,
             wm_ref, bm_ref, rs_ref, g2_ref, bb2_ref, o_ref):
        nb = n_ref[0]
        cfb = cf_ref[0]
        g1 = g1_ref[0]
        w1 = w1_ref[...]

        def expand(x):
            return jnp.reshape(
                jnp.broadcast_to(x[:, None, :], (mt, _K, x.shape[1])),
                (rows, x.shape[1]))

        s_c = jnp.sum(cfb, axis=1, keepdims=True)
        q_c = jnp.sum(cfb * cfb, axis=1, keepdims=True)
        s_n = jnp.sum(nb, axis=1, keepdims=True)
        q_n = jnp.sum(nb * nb, axis=1, keepdims=True)
        mu = (expand(s_c) + s_n) * jnp.float32(1.0 / (2 * _C))
        var = (expand(q_c) + q_n) * jnp.float32(1.0 / (2 * _C)) - mu * mu
        inv = lax.rsqrt(var + 1e-5)

        t1 = jnp.dot(cfb * g1[None, :_C], w1[:_C, :],
                     preferred_element_type=jnp.float32)
        t2 = jnp.dot(nb * g1[None, _C:], w1[_C:, :],
                     preferred_element_type=jnp.float32)
        gw1 = jnp.dot(g1[None, :], w1,
                      preferred_element_type=jnp.float32)
        bw1 = jnp.dot(bb1_ref[0][None, :], w1,
                      preferred_element_type=jnp.float32)
        h = jnp.maximum(
            inv * (expand(t1) + t2 - mu * gw1) + (bw1 + b1_ref[0]), 0.0)
        cw = jax.nn.sigmoid(
            jnp.dot(h, w2_ref[...], preferred_element_type=jnp.float32)
            + b2_ref[0])
        w = nb * cw
        wm = jnp.mean(jnp.reshape(w, (mt, _K, _C)), axis=1)
        fused = cfb + wm * rs_ref[0, 0]
        o = jnp.maximum(
            jnp.dot(fused, wm_ref[...], preferred_element_type=jnp.float32)
            + bm_ref[0], 0.0)
        mu2 = jnp.mean(o, axis=1, keepdims=True)
        var2 = jnp.mean((o - mu2) ** 2, axis=1, keepdims=True)
        o_ref[0] = (o - mu2) / jnp.sqrt(var2 + 1e-5) * g2_ref[0] + bb2_ref[0]

    def full(shape):
        return pl.BlockSpec(shape, lambda b, i: tuple(0 for _ in shape))

    nb = cf.shape[0]
    return pl.pallas_call(
        body,
        grid=(nb, _M // mt),
        in_specs=[
            pl.BlockSpec((1, rows, _C), lambda b, i: (b, i, 0)),
            pl.BlockSpec((1, mt, _C), lambda b, i: (b, i, 0)),
            full((1, 2 * _C)),
            full((1, 2 * _C)),
            full((2 * _C, _C)),
            full((1, _C)),
            full((_C, _C)),
            full((1, _C)),
            full((_C, _O)),
            full((1, _O)),
            full((1, 1)),
            full((1, _O)),
            full((1, _O)),
        ],
        out_specs=pl.BlockSpec((1, mt, _O), lambda b, i: (b, i, 0)),
        out_shape=jax.ShapeDtypeStruct((nb, _M, _O), jnp.float32),
    )(neigh_rows, cf, ln1_g.reshape(1, -1), ln1_b.reshape(1, -1), W1,
      b1.reshape(1, -1), W2, b2.reshape(1, -1), Wm, bm.reshape(1, -1),
      res_scale.reshape(1, 1), ln2_g.reshape(1, -1), ln2_b.reshape(1, -1))


def kernel(points, feats, center_idx, ln1_g, ln1_b, W1, b1, W2, b2, Wm, bm,
           res_scale, ln2_g, ln2_b):
    points_pad = jnp.pad(points, ((0, 0), (0, 0), (0, _PD - 3)))
    points_pad128 = jnp.pad(points, ((0, 0), (0, 0), (0, 128 - 3)))
    points_pad2d = points_pad128.reshape(_B * _N, 128)
    feats2d = feats.reshape(_B * _N, _C)
    ci_flat = center_idx.reshape(_B * _M)

    centers_rows = _gather_rows_sc(points_pad2d, ci_flat, 128, _M)
    cf_rows = _gather_rows_sc(feats2d, ci_flat, _C, _M)
    centers_pad = centers_rows.reshape(_B, _M, 128)[:, :, :_PD]
    cf = cf_rows.reshape(_B, _M, _C)
    points_t = jnp.transpose(points_pad, (0, 2, 1))

    hb = _B // 2
    outs, knns = [], []
    for h in range(2):
        sl = slice(h * hb, (h + 1) * hb)
        knn_idx = _tc_knn(centers_pad[sl], points_t[sl])
        knn_flat = knn_idx.reshape(hb * _M * _K)
        neigh_rows = _gather_rows_sc(feats2d, knn_flat, _C, _M * _K,
                                     batch0=h * hb)
        neigh = neigh_rows.reshape(hb, _M * _K, _C)
        out_h = _tc_mlp(neigh, cf[sl], ln1_g, ln1_b, W1, b1, W2, b2, Wm, bm,
                        res_scale, ln2_g, ln2_b)
        outs.append(out_h)
        knns.append(knn_idx)
    return (jnp.concatenate(outs, axis=0), jnp.concatenate(knns, axis=0))
